# Initial kernel scaffold; baseline (speedup 1.0000x reference)
import jax
import jax.numpy as jnp
from jax.experimental import pallas as pl


def kernel(memory, mem_coor, q_coor, B, Wq1, bq1, Wq2, bq2, Wk1, bk1, Wk2, bk2):
    raise NotImplementedError("write your pallas kernel here")


# speedup vs baseline: 4.5812x; 4.5812x over previous
import functools
import math

import jax
import jax.numpy as jnp
import numpy as np
from jax import lax
from jax.experimental import pallas as pl
from jax.experimental.pallas import tpu as pltpu
from jax.experimental.pallas import tpu_sc as plsc

EMB = 128
NBR = 16
LMEM = 8192
SQ = 2048
NPF = 64
_INV_SQRT_D = 1.0 / math.sqrt(EMB)

_dim_t = 10000.0 ** (2.0 * np.floor(np.arange(NPF) / 2.0) / NPF)
_FREQ = np.tile(2.0 * np.pi / _dim_t, 2).reshape(1, 2 * NPF).astype(np.float32)
_PHASE = np.tile(np.where(np.arange(NPF) % 2 == 0, 0.0, np.pi / 2.0), 2)
_PHASE = _PHASE.reshape(1, 2 * NPF).astype(np.float32)


def _posemb_mlp_body(coor_ref, freq_ref, phase_ref, w1t_ref, b1_ref,
                     w2t_ref, b2_ref, out_ref):
    cx = coor_ref[:, 0:1]
    cy = coor_ref[:, 1:2]
    lane = lax.broadcasted_iota(jnp.int32, (1, EMB), 1)
    coord = jnp.where(lane < NPF, cy, cx)
    emb = jnp.sin(coord * freq_ref[:, :] + phase_ref[:, :])
    h = jnp.maximum(
        jnp.dot(emb, w1t_ref[:, :], preferred_element_type=jnp.float32)
        + b1_ref[:, :], 0.0)
    out_ref[:, :] = (
        jnp.dot(h, w2t_ref[:, :], preferred_element_type=jnp.float32)
        + b2_ref[:, :])


def _posemb_mlp_add_body(coor_ref, freq_ref, phase_ref, w1t_ref, b1_ref,
                         w2t_ref, b2_ref, mem_ref, out_ref):
    cx = coor_ref[:, 0:1]
    cy = coor_ref[:, 1:2]
    lane = lax.broadcasted_iota(jnp.int32, (1, EMB), 1)
    coord = jnp.where(lane < NPF, cy, cx)
    emb = jnp.sin(coord * freq_ref[:, :] + phase_ref[:, :])
    h = jnp.maximum(
        jnp.dot(emb, w1t_ref[:, :], preferred_element_type=jnp.float32)
        + b1_ref[:, :], 0.0)
    out_ref[:, :] = (
        jnp.dot(h, w2t_ref[:, :], preferred_element_type=jnp.float32)
        + b2_ref[:, :] + mem_ref[:, :])


def _run_mlp(coor2, W1, b1, W2, b2, add=None, block=256):
    n = coor2.shape[0]
    full = lambda s: pl.BlockSpec(s, lambda i: (0, 0))
    in_specs = [
        pl.BlockSpec((block, 2), lambda i: (i, 0)),
        full((1, EMB)), full((1, EMB)),
        full((EMB, EMB)), full((1, EMB)),
        full((EMB, EMB)), full((1, EMB)),
    ]
    args = [coor2, jnp.asarray(_FREQ), jnp.asarray(_PHASE),
            W1.T, b1.reshape(1, EMB), W2.T, b2.reshape(1, EMB)]
    body = _posemb_mlp_body
    if add is not None:
        in_specs.append(pl.BlockSpec((block, EMB), lambda i: (i, 0)))
        args.append(add)
        body = _posemb_mlp_add_body
    return pl.pallas_call(
        body,
        grid=(n // block,),
        in_specs=in_specs,
        out_specs=pl.BlockSpec((block, EMB), lambda i: (i, 0)),
        out_shape=jax.ShapeDtypeStruct((n, EMB), jnp.float32),
    )(*args)


def _topk_body(qc_ref, mcT_ref, out_ref):
    qx = qc_ref[:, 0:1]
    qy = qc_ref[:, 1:2]
    mx = mcT_ref[0:1, :]
    my = mcT_ref[1:2, :]
    dx = qx - mx
    dy = qy - my
    d = jnp.sqrt(dx * dx + dy * dy)
    blk = d.shape[0]
    iota = lax.broadcasted_iota(jnp.int32, (blk, LMEM), 1)
    cols = []
    for _ in range(NBR):
        rowmin = jnp.min(d, axis=1, keepdims=True)
        idx = jnp.min(jnp.where(d == rowmin, iota, LMEM), axis=1,
                      keepdims=True)
        cols.append(idx)
        d = jnp.where(iota == idx, jnp.inf, d)
    out_ref[:, :] = jnp.concatenate(cols, axis=1)


def _run_topk(qc, mcT, block=64):
    return pl.pallas_call(
        _topk_body,
        grid=(SQ // block,),
        in_specs=[
            pl.BlockSpec((block, 2), lambda i: (i, 0)),
            pl.BlockSpec((2, LMEM), lambda i: (0, 0)),
        ],
        out_specs=pl.BlockSpec((block, NBR), lambda i: (i, 0)),
        out_shape=jax.ShapeDtypeStruct((SQ, NBR), jnp.int32),
    )(qc, mcT)


_NC, _NS = 2, 16
_NW = _NC * _NS
_QPW = SQ // _NW
_QPC = 8
_PPC = _QPC * NBR
_NCHUNK = _QPW // _QPC


def _xlane(v, op):
    lanes = lax.iota(jnp.int32, 16)
    for s in (1, 2, 4, 8):
        perm = jnp.bitwise_xor(lanes, s)
        v = op(v, v.at[perm].get(mode="promise_in_bounds"))
    return v


def _sc_combine_body(idx_hbm, kvpe_hbm, mem_hbm, qpos_hbm, out_hbm,
                     idx_v, kv_v, mv_v, q_v, o_v, sem1, sem2):
    wid = lax.axis_index("s") * _NC + lax.axis_index("c")

    def chunk(j, carry):
        pair_base = wid * (_QPW * NBR) + j * _PPC
        q_base = wid * _QPW + j * _QPC
        pltpu.sync_copy(idx_hbm.at[pl.ds(pair_base, _PPC)], idx_v)
        cp1 = pltpu.async_copy(kvpe_hbm.at[idx_v], kv_v, sem1)
        cp2 = pltpu.async_copy(mem_hbm.at[idx_v], mv_v, sem2)
        pltpu.sync_copy(qpos_hbm.at[pl.ds(q_base, _QPC)], q_v)
        cp1.wait()
        cp2.wait()

        def per_query(qq, c2):
            qvecs = [q_v[qq, pl.ds(16 * c, 16)] * _INV_SQRT_D
                     for c in range(8)]

            def per_pair(nn, c3):
                p = qq * NBR + nn
                logit = [qvecs[c] * kv_v[p, pl.ds(16 * c, 16)]
                         for c in range(8)]
                m = logit[0]
                for c in range(1, 8):
                    m = jnp.maximum(m, logit[c])
                mvec = _xlane(m, jnp.maximum)
                e = [jnp.exp(lg - mvec) for lg in logit]
                s = e[0]
                for c in range(1, 8):
                    s = s + e[c]
                inv = 1.0 / _xlane(s, jnp.add)
                for c in range(8):
                    o_v[p, pl.ds(16 * c, 16)] = (
                        (e[c] * inv) * mv_v[p, pl.ds(16 * c, 16)])
                return c3

            lax.fori_loop(0, NBR, per_pair, 0)
            return c2

        lax.fori_loop(0, _QPC, per_query, 0)
        pltpu.sync_copy(o_v, out_hbm.at[pl.ds(pair_base, _PPC)])
        return carry

    lax.fori_loop(0, _NCHUNK, chunk, 0)


@functools.lru_cache(maxsize=1)
def _sc_combine():
    return pl.kernel(
        _sc_combine_body,
        out_type=jax.ShapeDtypeStruct((SQ * NBR, EMB), jnp.float32),
        mesh=plsc.VectorSubcoreMesh(core_axis_name="c", subcore_axis_name="s",
                                    num_cores=_NC, num_subcores=_NS),
        scratch_types=[
            pltpu.VMEM((_PPC,), jnp.int32),
            pltpu.VMEM((_PPC, EMB), jnp.float32),
            pltpu.VMEM((_PPC, EMB), jnp.float32),
            pltpu.VMEM((_QPC, EMB), jnp.float32),
            pltpu.VMEM((_PPC, EMB), jnp.float32),
            pltpu.SemaphoreType.DMA,
            pltpu.SemaphoreType.DMA,
        ],
    )


def kernel(memory, mem_coor, q_coor, B, Wq1, bq1, Wq2, bq2, Wk1, bk1, Wk2, bk2):
    qc = q_coor[:, 1:3]
    mc = mem_coor[:, 1:3]
    query_pos = _run_mlp(qc, Wq1, bq1, Wq2, bq2)
    kv_pe = _run_mlp(mc, Wk1, bk1, Wk2, bk2, add=memory)
    topk = _run_topk(qc, mc.T)
    idx_flat = topk.reshape(-1)
    out_flat = _sc_combine()(idx_flat, kv_pe, memory, query_pos)
    return out_flat.reshape(1, SQ, NBR, EMB)


# speedup vs baseline: 4.7211x; 1.0305x over previous
import functools
import math

import jax
import jax.numpy as jnp
import numpy as np
from jax import lax
from jax.experimental import pallas as pl
from jax.experimental.pallas import tpu as pltpu
from jax.experimental.pallas import tpu_sc as plsc

EMB = 128
NBR = 16
LMEM = 8192
SQ = 2048
NPF = 64
_INV_SQRT_D = 1.0 / math.sqrt(EMB)

_dim_t = 10000.0 ** (2.0 * np.floor(np.arange(NPF) / 2.0) / NPF)
_FREQ = np.tile(2.0 * np.pi / _dim_t, 2).reshape(1, 2 * NPF).astype(np.float32)
_PHASE = np.tile(np.where(np.arange(NPF) % 2 == 0, 0.0, np.pi / 2.0), 2)
_PHASE = _PHASE.reshape(1, 2 * NPF).astype(np.float32)


def _posemb_mlp_body(coor_ref, freq_ref, phase_ref, w1t_ref, b1_ref,
                     w2t_ref, b2_ref, out_ref):
    cx = coor_ref[:, 0:1]
    cy = coor_ref[:, 1:2]
    lane = lax.broadcasted_iota(jnp.int32, (1, EMB), 1)
    coord = jnp.where(lane < NPF, cy, cx)
    emb = jnp.sin(coord * freq_ref[:, :] + phase_ref[:, :])
    h = jnp.maximum(
        jnp.dot(emb, w1t_ref[:, :], preferred_element_type=jnp.float32)
        + b1_ref[:, :], 0.0)
    out_ref[:, :] = (
        jnp.dot(h, w2t_ref[:, :], preferred_element_type=jnp.float32)
        + b2_ref[:, :])


def _posemb_mlp_add_body(coor_ref, freq_ref, phase_ref, w1t_ref, b1_ref,
                         w2t_ref, b2_ref, mem_ref, out_ref):
    cx = coor_ref[:, 0:1]
    cy = coor_ref[:, 1:2]
    lane = lax.broadcasted_iota(jnp.int32, (1, EMB), 1)
    coord = jnp.where(lane < NPF, cy, cx)
    emb = jnp.sin(coord * freq_ref[:, :] + phase_ref[:, :])
    h = jnp.maximum(
        jnp.dot(emb, w1t_ref[:, :], preferred_element_type=jnp.float32)
        + b1_ref[:, :], 0.0)
    out_ref[:, :] = (
        jnp.dot(h, w2t_ref[:, :], preferred_element_type=jnp.float32)
        + b2_ref[:, :] + mem_ref[:, :])


def _run_mlp(coor2, W1, b1, W2, b2, add=None, block=256):
    n = coor2.shape[0]
    full = lambda s: pl.BlockSpec(s, lambda i: (0, 0))
    in_specs = [
        pl.BlockSpec((block, 2), lambda i: (i, 0)),
        full((1, EMB)), full((1, EMB)),
        full((EMB, EMB)), full((1, EMB)),
        full((EMB, EMB)), full((1, EMB)),
    ]
    args = [coor2, jnp.asarray(_FREQ), jnp.asarray(_PHASE),
            W1.T, b1.reshape(1, EMB), W2.T, b2.reshape(1, EMB)]
    body = _posemb_mlp_body
    if add is not None:
        in_specs.append(pl.BlockSpec((block, EMB), lambda i: (i, 0)))
        args.append(add)
        body = _posemb_mlp_add_body
    return pl.pallas_call(
        body,
        grid=(n // block,),
        in_specs=in_specs,
        out_specs=pl.BlockSpec((block, EMB), lambda i: (i, 0)),
        out_shape=jax.ShapeDtypeStruct((n, EMB), jnp.float32),
    )(*args)


def _topk_body(qc_ref, mcT_ref, out_ref):
    qx = qc_ref[:, 0:1]
    qy = qc_ref[:, 1:2]
    mx = mcT_ref[0:1, :]
    my = mcT_ref[1:2, :]
    dx = qx - mx
    dy = qy - my
    d = jnp.sqrt(dx * dx + dy * dy)
    blk = d.shape[0]
    iota = lax.broadcasted_iota(jnp.int32, (blk, LMEM), 1)
    cols = []
    for _ in range(NBR):
        rowmin = jnp.min(d, axis=1, keepdims=True)
        idx = jnp.min(jnp.where(d == rowmin, iota, LMEM), axis=1,
                      keepdims=True)
        cols.append(idx)
        d = jnp.where(iota == idx, jnp.inf, d)
    out_ref[:, :] = jnp.concatenate(cols, axis=1)


def _run_topk(qc, mcT, block=64):
    return pl.pallas_call(
        _topk_body,
        grid=(SQ // block,),
        in_specs=[
            pl.BlockSpec((block, 2), lambda i: (i, 0)),
            pl.BlockSpec((2, LMEM), lambda i: (0, 0)),
        ],
        out_specs=pl.BlockSpec((block, NBR), lambda i: (i, 0)),
        out_shape=jax.ShapeDtypeStruct((SQ, NBR), jnp.int32),
    )(qc, mcT)


_NC, _NS = 2, 16
_NW = _NC * _NS
_QPW = SQ // _NW
_QPC = 8
_PPC = _QPC * NBR
_NCHUNK = _QPW // _QPC


def _xlane(v, op):
    lanes = lax.iota(jnp.int32, 16)
    for s in (1, 2, 4, 8):
        perm = jnp.bitwise_xor(lanes, s)
        v = op(v, v.at[perm].get(mode="promise_in_bounds"))
    return v


def _sc_combine_body(idx_hbm, kvpe_hbm, mem_hbm, qpos_hbm, out_hbm,
                     idx_v0, idx_v1, kv_v0, kv_v1, mv_v0, mv_v1,
                     o_v0, o_v1, q_v,
                     semk0, semk1, semm0, semm1, semo0, semo1):
    wid = lax.axis_index("s") * _NC + lax.axis_index("c")
    idx_vs = (idx_v0, idx_v1)
    kv_vs = (kv_v0, kv_v1)
    mv_vs = (mv_v0, mv_v1)
    o_vs = (o_v0, o_v1)
    semk = (semk0, semk1)
    semm = (semm0, semm1)
    semo = (semo0, semo1)

    pltpu.sync_copy(qpos_hbm.at[pl.ds(wid * _QPW, _QPW)], q_v)

    def compute_chunk(j, slot):
        kv_v, mv_v, o_v = kv_vs[slot], mv_vs[slot], o_vs[slot]

        def per_query(qq, c2):
            qvecs = [q_v[j * _QPC + qq, pl.ds(16 * c, 16)] * _INV_SQRT_D
                     for c in range(8)]

            def per_pair(nn, c3):
                p = qq * NBR + nn
                logit = [qvecs[c] * kv_v[p, pl.ds(16 * c, 16)]
                         for c in range(8)]
                m = logit[0]
                for c in range(1, 8):
                    m = jnp.maximum(m, logit[c])
                mvec = _xlane(m, jnp.maximum)
                e = [jnp.exp(lg - mvec) for lg in logit]
                s = e[0]
                for c in range(1, 8):
                    s = s + e[c]
                inv = 1.0 / _xlane(s, jnp.add)
                for c in range(8):
                    o_v[p, pl.ds(16 * c, 16)] = (
                        (e[c] * inv) * mv_v[p, pl.ds(16 * c, 16)])
                return c3

            lax.fori_loop(0, NBR, per_pair, 0)
            return c2

        lax.fori_loop(0, _QPC, per_query, 0)

    pend = [None, None]
    outp = [None, None]
    for j in range(_NCHUNK + 1):
        slot = j & 1
        if j < _NCHUNK:
            base = wid * (_QPW * NBR) + j * _PPC
            pltpu.sync_copy(idx_hbm.at[pl.ds(base, _PPC)], idx_vs[slot])
            pend[slot] = (
                pltpu.async_copy(kvpe_hbm.at[idx_vs[slot]], kv_vs[slot],
                                 semk[slot]),
                pltpu.async_copy(mem_hbm.at[idx_vs[slot]], mv_vs[slot],
                                 semm[slot]),
            )
        if j >= 1:
            pslot = 1 - slot
            pj = j - 1
            c1, c2 = pend[pslot]
            c1.wait()
            c2.wait()
            if outp[pslot] is not None:
                outp[pslot].wait()
            compute_chunk(pj, pslot)
            pbase = wid * (_QPW * NBR) + pj * _PPC
            outp[pslot] = pltpu.async_copy(
                o_vs[pslot], out_hbm.at[pl.ds(pbase, _PPC)], semo[pslot])
    outp[0].wait()
    outp[1].wait()


@functools.lru_cache(maxsize=1)
def _sc_combine():
    return pl.kernel(
        _sc_combine_body,
        out_type=jax.ShapeDtypeStruct((SQ * NBR, EMB), jnp.float32),
        mesh=plsc.VectorSubcoreMesh(core_axis_name="c", subcore_axis_name="s",
                                    num_cores=_NC, num_subcores=_NS),
        scratch_types=(
            [pltpu.VMEM((_PPC,), jnp.int32)] * 2
            + [pltpu.VMEM((_PPC, EMB), jnp.float32)] * 6
            + [pltpu.VMEM((_QPW, EMB), jnp.float32)]
            + [pltpu.SemaphoreType.DMA] * 6
        ),
    )


def kernel(memory, mem_coor, q_coor, B, Wq1, bq1, Wq2, bq2, Wk1, bk1, Wk2, bk2):
    qc = q_coor[:, 1:3]
    mc = mem_coor[:, 1:3]
    query_pos = _run_mlp(qc, Wq1, bq1, Wq2, bq2)
    kv_pe = _run_mlp(mc, Wk1, bk1, Wk2, bk2, add=memory)
    topk = _run_topk(qc, mc.T)
    idx_flat = topk.reshape(-1)
    out_flat = _sc_combine()(idx_flat, kv_pe, memory, query_pos)
    return out_flat.reshape(1, SQ, NBR, EMB)


# speedup vs baseline: 4.9708x; 1.0529x over previous
import functools
import math

import jax
import jax.numpy as jnp
import numpy as np
from jax import lax
from jax.experimental import pallas as pl
from jax.experimental.pallas import tpu as pltpu
from jax.experimental.pallas import tpu_sc as plsc

EMB = 128
NBR = 16
LMEM = 8192
SQ = 2048
NPF = 64
_INV_SQRT_D = 1.0 / math.sqrt(EMB)

_dim_t = 10000.0 ** (2.0 * np.floor(np.arange(NPF) / 2.0) / NPF)
_FREQ = np.tile(2.0 * np.pi / _dim_t, 2).reshape(1, 2 * NPF).astype(np.float32)
_PHASE = np.tile(np.where(np.arange(NPF) % 2 == 0, 0.0, np.pi / 2.0), 2)
_PHASE = _PHASE.reshape(1, 2 * NPF).astype(np.float32)


def _posemb_mlp_body(coor_ref, freq_ref, phase_ref, w1t_ref, b1_ref,
                     w2t_ref, b2_ref, out_ref):
    cx = coor_ref[:, 0:1]
    cy = coor_ref[:, 1:2]
    lane = lax.broadcasted_iota(jnp.int32, (1, EMB), 1)
    coord = jnp.where(lane < NPF, cy, cx)
    emb = jnp.sin(coord * freq_ref[:, :] + phase_ref[:, :])
    h = jnp.maximum(
        jnp.dot(emb, w1t_ref[:, :], preferred_element_type=jnp.float32)
        + b1_ref[:, :], 0.0)
    out_ref[:, :] = (
        jnp.dot(h, w2t_ref[:, :], preferred_element_type=jnp.float32)
        + b2_ref[:, :])


def _posemb_mlp_add_body(coor_ref, freq_ref, phase_ref, w1t_ref, b1_ref,
                         w2t_ref, b2_ref, mem_ref, out_ref):
    cx = coor_ref[:, 0:1]
    cy = coor_ref[:, 1:2]
    lane = lax.broadcasted_iota(jnp.int32, (1, EMB), 1)
    coord = jnp.where(lane < NPF, cy, cx)
    emb = jnp.sin(coord * freq_ref[:, :] + phase_ref[:, :])
    h = jnp.maximum(
        jnp.dot(emb, w1t_ref[:, :], preferred_element_type=jnp.float32)
        + b1_ref[:, :], 0.0)
    out_ref[:, :] = (
        jnp.dot(h, w2t_ref[:, :], preferred_element_type=jnp.float32)
        + b2_ref[:, :] + mem_ref[:, :])


def _run_mlp(coor2, W1, b1, W2, b2, add=None, block=256):
    n = coor2.shape[0]
    full = lambda s: pl.BlockSpec(s, lambda i: (0, 0))
    in_specs = [
        pl.BlockSpec((block, 2), lambda i: (i, 0)),
        full((1, EMB)), full((1, EMB)),
        full((EMB, EMB)), full((1, EMB)),
        full((EMB, EMB)), full((1, EMB)),
    ]
    args = [coor2, jnp.asarray(_FREQ), jnp.asarray(_PHASE),
            W1.T, b1.reshape(1, EMB), W2.T, b2.reshape(1, EMB)]
    body = _posemb_mlp_body
    if add is not None:
        in_specs.append(pl.BlockSpec((block, EMB), lambda i: (i, 0)))
        args.append(add)
        body = _posemb_mlp_add_body
    return pl.pallas_call(
        body,
        grid=(n // block,),
        in_specs=in_specs,
        out_specs=pl.BlockSpec((block, EMB), lambda i: (i, 0)),
        out_shape=jax.ShapeDtypeStruct((n, EMB), jnp.float32),
    )(*args)


def _topk_body(qc_ref, mcT_ref, out_ref):
    qx = qc_ref[:, 0:1]
    qy = qc_ref[:, 1:2]
    mx = mcT_ref[0:1, :]
    my = mcT_ref[1:2, :]
    dx = qx - mx
    dy = qy - my
    d = jnp.sqrt(dx * dx + dy * dy)
    blk = d.shape[0]
    iota = lax.broadcasted_iota(jnp.int32, (blk, LMEM), 1)
    cols = []
    for _ in range(NBR):
        rowmin = jnp.min(d, axis=1, keepdims=True)
        idx = jnp.min(jnp.where(d == rowmin, iota, LMEM), axis=1,
                      keepdims=True)
        cols.append(idx)
        d = jnp.where(iota == idx, jnp.inf, d)
    out_ref[:, :] = jnp.concatenate(cols, axis=1)


def _run_topk(qc, mcT, block=64):
    n = qc.shape[0]
    return pl.pallas_call(
        _topk_body,
        grid=(n // block,),
        in_specs=[
            pl.BlockSpec((block, 2), lambda i: (i, 0)),
            pl.BlockSpec((2, LMEM), lambda i: (0, 0)),
        ],
        out_specs=pl.BlockSpec((block, NBR), lambda i: (i, 0)),
        out_shape=jax.ShapeDtypeStruct((n, NBR), jnp.int32),
    )(qc, mcT)


_NC, _NS = 2, 16
_NW = _NC * _NS
_QPC = 8
_PPC = _QPC * NBR


def _xlane(v, op):
    lanes = lax.iota(jnp.int32, 16)
    for s in (1, 2, 4, 8):
        perm = jnp.bitwise_xor(lanes, s)
        v = op(v, v.at[perm].get(mode="promise_in_bounds"))
    return v


def _sc_combine_body(idx_hbm, kvpe_hbm, mem_hbm, qpos_hbm, out_hbm,
                     idx_v0, idx_v1, kv_v0, kv_v1, mv_v0, mv_v1,
                     o_v0, o_v1, q_v,
                     semk0, semk1, semm0, semm1, semo0, semo1,
                     *, _QPW, _NCHUNK):
    wid = lax.axis_index("s") * _NC + lax.axis_index("c")
    idx_vs = (idx_v0, idx_v1)
    kv_vs = (kv_v0, kv_v1)
    mv_vs = (mv_v0, mv_v1)
    o_vs = (o_v0, o_v1)
    semk = (semk0, semk1)
    semm = (semm0, semm1)
    semo = (semo0, semo1)

    pltpu.sync_copy(qpos_hbm.at[pl.ds(wid * _QPW, _QPW)], q_v)

    def compute_chunk(j, slot):
        kv_v, mv_v, o_v = kv_vs[slot], mv_vs[slot], o_vs[slot]

        def per_query(qq, c2):
            qvecs = [q_v[j * _QPC + qq, pl.ds(16 * c, 16)] * _INV_SQRT_D
                     for c in range(8)]

            def per_pair(nn, c3):
                p = qq * NBR + nn
                logit = [qvecs[c] * kv_v[p, pl.ds(16 * c, 16)]
                         for c in range(8)]
                m = logit[0]
                for c in range(1, 8):
                    m = jnp.maximum(m, logit[c])
                mvec = _xlane(m, jnp.maximum)
                e = [jnp.exp(lg - mvec) for lg in logit]
                s = e[0]
                for c in range(1, 8):
                    s = s + e[c]
                inv = 1.0 / _xlane(s, jnp.add)
                for c in range(8):
                    o_v[p, pl.ds(16 * c, 16)] = (
                        (e[c] * inv) * mv_v[p, pl.ds(16 * c, 16)])
                return c3

            lax.fori_loop(0, NBR, per_pair, 0)
            return c2

        lax.fori_loop(0, _QPC, per_query, 0)

    pend = [None, None]
    outp = [None, None]
    for j in range(_NCHUNK + 1):
        slot = j & 1
        if j < _NCHUNK:
            base = wid * (_QPW * NBR) + j * _PPC
            pltpu.sync_copy(idx_hbm.at[pl.ds(base, _PPC)], idx_vs[slot])
            pend[slot] = (
                pltpu.async_copy(kvpe_hbm.at[idx_vs[slot]], kv_vs[slot],
                                 semk[slot]),
                pltpu.async_copy(mem_hbm.at[idx_vs[slot]], mv_vs[slot],
                                 semm[slot]),
            )
        if j >= 1:
            pslot = 1 - slot
            pj = j - 1
            c1, c2 = pend[pslot]
            c1.wait()
            c2.wait()
            if outp[pslot] is not None:
                outp[pslot].wait()
            compute_chunk(pj, pslot)
            pbase = wid * (_QPW * NBR) + pj * _PPC
            outp[pslot] = pltpu.async_copy(
                o_vs[pslot], out_hbm.at[pl.ds(pbase, _PPC)], semo[pslot])
    outp[0].wait()
    outp[1].wait()


@functools.lru_cache(maxsize=4)
def _sc_combine(nq):
    qpw = nq // _NW
    nchunk = qpw // _QPC
    body = functools.partial(_sc_combine_body, _QPW=qpw, _NCHUNK=nchunk)
    return pl.kernel(
        body,
        out_type=jax.ShapeDtypeStruct((nq * NBR, EMB), jnp.float32),
        mesh=plsc.VectorSubcoreMesh(core_axis_name="c", subcore_axis_name="s",
                                    num_cores=_NC, num_subcores=_NS),
        scratch_types=(
            [pltpu.VMEM((_PPC,), jnp.int32)] * 2
            + [pltpu.VMEM((_PPC, EMB), jnp.float32)] * 6
            + [pltpu.VMEM((qpw, EMB), jnp.float32)]
            + [pltpu.SemaphoreType.DMA] * 6
        ),
    )


def kernel(memory, mem_coor, q_coor, B, Wq1, bq1, Wq2, bq2, Wk1, bk1, Wk2, bk2):
    qc = q_coor[:, 1:3]
    mc = mem_coor[:, 1:3]
    query_pos = _run_mlp(qc, Wq1, bq1, Wq2, bq2)
    kv_pe = _run_mlp(mc, Wk1, bk1, Wk2, bk2, add=memory)
    half = SQ // 2
    mcT = mc.T
    outs = []
    for h in range(2):
        qch = qc[h * half:(h + 1) * half]
        qph = query_pos[h * half:(h + 1) * half]
        topk = _run_topk(qch, mcT)
        outs.append(_sc_combine(half)(topk.reshape(-1), kv_pe, memory, qph))
    return jnp.concatenate(outs, axis=0).reshape(1, SQ, NBR, EMB)
